# Initial kernel scaffold; baseline (speedup 1.0000x reference)
import jax
import jax.numpy as jnp
from jax.experimental import pallas as pl


def kernel(x, edge_index, cluster_ids, W_pre, b_pre, W_ops, alphas, W_out):
    raise NotImplementedError("write your pallas kernel here")


# speedup vs baseline: 7.0098x; 7.0098x over previous
import functools

import jax
import jax.numpy as jnp
from jax import lax
from jax.experimental import pallas as pl
from jax.experimental.pallas import tpu as pltpu
from jax.experimental.pallas import tpu_sc as plsc

_N = 10000
_E = 320000
_D = 128
_H = 128
_HH = _H // 2
_K = 4
_CLS = 8

_NC = 2
_NS = 16
_C = 125
_EPT = _E // _NS
_CH = _EPT // _C
_NP = 10240
_RPT = _NP // _NS
_ROWS_TOTAL = _E // _C


def _pre_tc(x, W_pre, b_pre):
    def body(x_ref, w_ref, b_ref, o_ref):
        h = jnp.dot(x_ref[...], w_ref[...],
                    preferred_element_type=jnp.float32) + b_ref[...]
        o_ref[:_N, :] = h[:, :_HH]
        o_ref[_N:, :] = h[:, _HH:]

    return pl.pallas_call(
        body,
        out_shape=jax.ShapeDtypeStruct((2 * _N, _HH), jnp.float32),
    )(x, W_pre, b_pre.reshape(1, _H))


def _sc_agg(hstack, srcx, dst2, zD, z1, o1):
    mesh = plsc.VectorSubcoreMesh(core_axis_name="c", subcore_axis_name="s")

    @functools.partial(
        pl.kernel,
        out_type=(jax.ShapeDtypeStruct((2 * _NP, _HH), jnp.float32),
                  jax.ShapeDtypeStruct((2 * _NP, 16), jnp.float32)),
        mesh=mesh,
        compiler_params=pltpu.CompilerParams(use_tc_tiling_on_sc=False),
        scratch_types=[
            pltpu.VMEM((_CH, _C), jnp.int32),
            pltpu.VMEM((_CH, _C), jnp.int32),
            pltpu.VMEM((_C, _HH), jnp.float32),
            pltpu.VMEM((_C, 16), jnp.float32),
            pltpu.VMEM_SHARED((_NP, _HH), jnp.float32),
            pltpu.VMEM_SHARED((_NP, 16), jnp.float32),
            pltpu.SemaphoreType.DMA,
        ],
    )
    def body(h_hbm, src_hbm, dst_hbm, zD_hbm, z1_hbm, o1_hbm,
             agg_hbm, deg_hbm,
             src_v, dst_v, rows_v, ones_v, agg_sh, deg_sh, sem):
        c = lax.axis_index("c")
        s = lax.axis_index("s")

        pltpu.sync_copy(zD_hbm, agg_sh.at[pl.ds(s * _RPT, _RPT)])
        pltpu.sync_copy(z1_hbm, deg_sh.at[pl.ds(s * _RPT, _RPT)])
        pltpu.sync_copy(o1_hbm, ones_v)
        pltpu.sync_copy(src_hbm.at[pl.ds(c * _ROWS_TOTAL + s * _CH, _CH)],
                        src_v)
        pltpu.sync_copy(dst_hbm.at[pl.ds(s * _CH, _CH)], dst_v)
        plsc.subcore_barrier()

        def chunk(j, carry):
            pltpu.async_copy(h_hbm.at[src_v.at[j]], rows_v, sem).wait()
            pltpu.sync_copy(rows_v, agg_sh.at[dst_v.at[j]], add=True)

            @pl.when((j >= c * (_CH // 2)) & (j < (c + 1) * (_CH // 2)))
            def _():
                pltpu.sync_copy(ones_v, deg_sh.at[dst_v.at[j]], add=True)

            return carry

        lax.fori_loop(0, _CH, chunk, 0)
        plsc.subcore_barrier()

        base = c * _NP + s * _RPT
        pltpu.sync_copy(agg_sh.at[pl.ds(s * _RPT, _RPT)],
                        agg_hbm.at[pl.ds(base, _RPT)])
        pltpu.sync_copy(deg_sh.at[pl.ds(s * _RPT, _RPT)],
                        deg_hbm.at[pl.ds(base, _RPT)])

    return body(hstack, srcx, dst2, zD, z1, o1)


def _post_tc(agg2, deg2, cid, W_ops, alphas, W_out):
    def body(a_ref, d_ref, c_ref, w_ref, al_ref, wo_ref, o_ref):
        deg = jnp.maximum(d_ref[:_N, 0:1] + d_ref[_NP:_NP + _N, 0:1], 1.0)
        agg = jnp.concatenate(
            [a_ref[:_N, :], a_ref[_NP:_NP + _N, :]], axis=1) / deg
        cid = c_ref[...]
        res = jnp.zeros((_N, _H), jnp.float32)
        for k in range(_K):
            hk = jnp.dot(agg, w_ref[k], preferred_element_type=jnp.float32)
            mask = (cid == k).astype(jnp.float32)
            res = res + al_ref[0, k] * (mask * hk)
        o_ref[...] = jnp.dot(res, wo_ref[...],
                             preferred_element_type=jnp.float32)

    return pl.pallas_call(
        body,
        out_shape=jax.ShapeDtypeStruct((_N, _CLS), jnp.float32),
    )(agg2, deg2, cid, W_ops, alphas.reshape(1, _K), W_out)


def kernel(x, edge_index, cluster_ids, W_pre, b_pre, W_ops, alphas, W_out):
    x = x.astype(jnp.float32)
    ei = edge_index.astype(jnp.int32)
    src2 = ei[0].reshape(_ROWS_TOTAL, _C)
    srcx = jnp.concatenate([src2, src2 + _N], axis=0)
    dst2 = ei[1].reshape(_ROWS_TOTAL, _C)
    cid = cluster_ids.astype(jnp.int32).reshape(_N, 1)
    zD = jnp.zeros((_RPT, _HH), jnp.float32)
    z1 = jnp.zeros((_RPT, 16), jnp.float32)
    o1 = jnp.ones((_C, 16), jnp.float32)

    hstack = _pre_tc(x, W_pre, b_pre)
    agg2, deg2 = _sc_agg(hstack, srcx, dst2, zD, z1, o1)
    return _post_tc(agg2, deg2, cid, W_ops, alphas, W_out)


# speedup vs baseline: 10.4956x; 1.4973x over previous
import functools

import jax
import jax.numpy as jnp
from jax import lax
from jax.experimental import pallas as pl
from jax.experimental.pallas import tpu as pltpu
from jax.experimental.pallas import tpu_sc as plsc

_N = 10000
_E = 320000
_D = 128
_H = 128
_HH = _H // 2
_K = 4
_CLS = 8

_NC = 2
_NS = 16
_C = 125
_EPT = _E // _NS
_CH = _EPT // _C
_NP = 10240
_RPT = _NP // _NS
_ROWS_TOTAL = _E // _C


def _pre_tc(x, W_pre, b_pre):
    def body(x_ref, w_ref, b_ref, o_ref):
        h = jnp.dot(x_ref[...], w_ref[...],
                    preferred_element_type=jnp.float32) + b_ref[...]
        o_ref[:_N, :] = h[:, :_HH]
        o_ref[_N:, :] = h[:, _HH:]

    return pl.pallas_call(
        body,
        out_shape=jax.ShapeDtypeStruct((2 * _N, _HH), jnp.float32),
    )(x, W_pre, b_pre.reshape(1, _H))


def _sc_agg(hstack, srcx, dst2, zD, z1, o1):
    mesh = plsc.VectorSubcoreMesh(core_axis_name="c", subcore_axis_name="s")

    @functools.partial(
        pl.kernel,
        out_type=(jax.ShapeDtypeStruct((2 * _NP, _HH), jnp.float32),
                  jax.ShapeDtypeStruct((2 * _NP, 16), jnp.float32)),
        mesh=mesh,
        compiler_params=pltpu.CompilerParams(use_tc_tiling_on_sc=False),
        scratch_types=[
            pltpu.VMEM((_CH, _C), jnp.int32),
            pltpu.VMEM((_CH, _C), jnp.int32),
            pltpu.VMEM((_C, _HH), jnp.float32),
            pltpu.VMEM((_C, _HH), jnp.float32),
            pltpu.VMEM((_C, 16), jnp.float32),
            pltpu.VMEM_SHARED((_NP, _HH), jnp.float32),
            pltpu.VMEM_SHARED((_NP, 16), jnp.float32),
            pltpu.SemaphoreType.DMA,
            pltpu.SemaphoreType.DMA,
        ],
    )
    def body(h_hbm, src_hbm, dst_hbm, zD_hbm, z1_hbm, o1_hbm,
             agg_hbm, deg_hbm,
             src_v, dst_v, rows0_v, rows1_v, ones_v, agg_sh, deg_sh,
             sem0, sem1):
        c = lax.axis_index("c")
        s = lax.axis_index("s")

        pltpu.sync_copy(zD_hbm, agg_sh.at[pl.ds(s * _RPT, _RPT)])
        pltpu.sync_copy(z1_hbm, deg_sh.at[pl.ds(s * _RPT, _RPT)])
        pltpu.sync_copy(o1_hbm, ones_v)
        pltpu.sync_copy(src_hbm.at[pl.ds(c * _ROWS_TOTAL + s * _CH, _CH)],
                        src_v)
        pltpu.sync_copy(dst_hbm.at[pl.ds(s * _CH, _CH)], dst_v)
        plsc.subcore_barrier()

        def half_step(j, rows_b, sem_b):
            pltpu.make_async_copy(h_hbm.at[src_v.at[j]], rows_b, sem_b).wait()
            pltpu.sync_copy(rows_b, agg_sh.at[dst_v.at[j]], add=True)

            @pl.when(j + 2 < _CH)
            def _():
                pltpu.async_copy(h_hbm.at[src_v.at[j + 2]], rows_b, sem_b)

            @pl.when((j >= c * (_CH // 2)) & (j < (c + 1) * (_CH // 2)))
            def _():
                pltpu.sync_copy(ones_v, deg_sh.at[dst_v.at[j]], add=True)

        def chunk_pair(i, carry):
            half_step(2 * i, rows0_v, sem0)
            half_step(2 * i + 1, rows1_v, sem1)
            return carry

        pltpu.async_copy(h_hbm.at[src_v.at[0]], rows0_v, sem0)
        pltpu.async_copy(h_hbm.at[src_v.at[1]], rows1_v, sem1)
        lax.fori_loop(0, _CH // 2, chunk_pair, 0)
        plsc.subcore_barrier()

        base = c * _NP + s * _RPT
        pltpu.sync_copy(agg_sh.at[pl.ds(s * _RPT, _RPT)],
                        agg_hbm.at[pl.ds(base, _RPT)])
        pltpu.sync_copy(deg_sh.at[pl.ds(s * _RPT, _RPT)],
                        deg_hbm.at[pl.ds(base, _RPT)])

    return body(hstack, srcx, dst2, zD, z1, o1)


def _post_tc(agg2, deg2, cid, W_ops, alphas, W_out):
    def body(a_ref, d_ref, c_ref, w_ref, al_ref, wo_ref, o_ref):
        deg = jnp.maximum(d_ref[:_N, 0:1] + d_ref[_NP:_NP + _N, 0:1], 1.0)
        agg = jnp.concatenate(
            [a_ref[:_N, :], a_ref[_NP:_NP + _N, :]], axis=1) / deg
        cid = c_ref[...]
        res = jnp.zeros((_N, _H), jnp.float32)
        for k in range(_K):
            hk = jnp.dot(agg, w_ref[k], preferred_element_type=jnp.float32)
            mask = (cid == k).astype(jnp.float32)
            res = res + al_ref[0, k] * (mask * hk)
        o_ref[...] = jnp.dot(res, wo_ref[...],
                             preferred_element_type=jnp.float32)

    return pl.pallas_call(
        body,
        out_shape=jax.ShapeDtypeStruct((_N, _CLS), jnp.float32),
    )(agg2, deg2, cid, W_ops, alphas.reshape(1, _K), W_out)


def kernel(x, edge_index, cluster_ids, W_pre, b_pre, W_ops, alphas, W_out):
    x = x.astype(jnp.float32)
    ei = edge_index.astype(jnp.int32)
    src2 = ei[0].reshape(_ROWS_TOTAL, _C)
    srcx = jnp.concatenate([src2, src2 + _N], axis=0)
    dst2 = ei[1].reshape(_ROWS_TOTAL, _C)
    cid = cluster_ids.astype(jnp.int32).reshape(_N, 1)
    zD = jnp.zeros((_RPT, _HH), jnp.float32)
    z1 = jnp.zeros((_RPT, 16), jnp.float32)
    o1 = jnp.ones((_C, 16), jnp.float32)

    hstack = _pre_tc(x, W_pre, b_pre)
    agg2, deg2 = _sc_agg(hstack, srcx, dst2, zD, z1, o1)
    return _post_tc(agg2, deg2, cid, W_ops, alphas, W_out)


# speedup vs baseline: 10.6171x; 1.0116x over previous
import functools

import jax
import jax.numpy as jnp
from jax import lax
from jax.experimental import pallas as pl
from jax.experimental.pallas import tpu as pltpu
from jax.experimental.pallas import tpu_sc as plsc

_N = 10000
_E = 320000
_D = 128
_H = 128
_HH = _H // 2
_K = 4
_CLS = 8

_NC = 2
_NS = 16
_C = 125
_EPT = _E // _NS
_CH = _EPT // _C
_NP = 10240
_RPT = _NP // _NS
_ROWS_TOTAL = _E // _C


def _pre_tc(x, W_pre, b_pre):
    def body(x_ref, w_ref, b_ref, o_ref):
        h = jnp.dot(x_ref[...], w_ref[...],
                    preferred_element_type=jnp.float32) + b_ref[...]
        o_ref[:_N, :] = h[:, :_HH]
        o_ref[_N:, :] = h[:, _HH:]

    return pl.pallas_call(
        body,
        out_shape=jax.ShapeDtypeStruct((2 * _N, _HH), jnp.float32),
    )(x, W_pre, b_pre.reshape(1, _H))


def _sc_agg(hstack, srcx, dst2, zD, z1, o1):
    mesh = plsc.VectorSubcoreMesh(core_axis_name="c", subcore_axis_name="s")

    @functools.partial(
        pl.kernel,
        out_type=(jax.ShapeDtypeStruct((2 * _NP, _HH), jnp.float32),
                  jax.ShapeDtypeStruct((2 * _NP, 16), jnp.float32)),
        mesh=mesh,
        compiler_params=pltpu.CompilerParams(use_tc_tiling_on_sc=False),
        scratch_types=[
            pltpu.VMEM((_CH, _C), jnp.int32),
            pltpu.VMEM((_CH, _C), jnp.int32),
            pltpu.VMEM((_C, _HH), jnp.float32),
            pltpu.VMEM((_C, _HH), jnp.float32),
            pltpu.VMEM((_C, _HH), jnp.float32),
            pltpu.VMEM((_C, _HH), jnp.float32),
            pltpu.VMEM((_C, 16), jnp.float32),
            pltpu.VMEM_SHARED((_NP, _HH), jnp.float32),
            pltpu.VMEM_SHARED((_NP, 16), jnp.float32),
            [pltpu.SemaphoreType.DMA] * 4,
            [pltpu.SemaphoreType.DMA] * 4,
            pltpu.SemaphoreType.DMA,
        ],
    )
    def body(h_hbm, src_hbm, dst_hbm, zD_hbm, z1_hbm, o1_hbm,
             agg_hbm, deg_hbm,
             src_v, dst_v, rows0_v, rows1_v, rows2_v, rows3_v, ones_v,
             agg_sh, deg_sh, gsems, ssems, dsem):
        c = lax.axis_index("c")
        s = lax.axis_index("s")

        pltpu.sync_copy(zD_hbm, agg_sh.at[pl.ds(s * _RPT, _RPT)])
        pltpu.sync_copy(z1_hbm, deg_sh.at[pl.ds(s * _RPT, _RPT)])
        pltpu.sync_copy(o1_hbm, ones_v)
        pltpu.sync_copy(src_hbm.at[pl.ds(c * _ROWS_TOTAL + s * _CH, _CH)],
                        src_v)
        pltpu.sync_copy(dst_hbm.at[pl.ds(s * _CH, _CH)], dst_v)
        plsc.subcore_barrier()

        rows = [rows0_v, rows1_v, rows2_v, rows3_v]

        def step(j, b):
            rows_b = rows[b]
            pltpu.make_async_copy(h_hbm.at[src_v.at[j]], rows_b,
                                  gsems[b]).wait()
            pltpu.async_copy(rows_b, agg_sh.at[dst_v.at[j]], ssems[b],
                             add=True)

            @pl.when((j >= c * (_CH // 2)) & (j < (c + 1) * (_CH // 2)))
            def _():
                pltpu.async_copy(ones_v, deg_sh.at[dst_v.at[j]], dsem,
                                 add=True)

            b2 = (b + 2) % 4

            @pl.when((j >= 2) & (j + 2 < _CH))
            def _():
                pltpu.make_async_copy(rows[b2], agg_sh.at[dst_v.at[0]],
                                      ssems[b2]).wait()

            @pl.when(j + 2 < _CH)
            def _():
                pltpu.async_copy(h_hbm.at[src_v.at[j + 2]], rows[b2],
                                 gsems[b2])

        def chunk_quad(i, carry):
            for b in range(4):
                step(4 * i + b, b)
            return carry

        pltpu.async_copy(h_hbm.at[src_v.at[0]], rows0_v, gsems[0])
        pltpu.async_copy(h_hbm.at[src_v.at[1]], rows1_v, gsems[1])
        lax.fori_loop(0, _CH // 4, chunk_quad, 0)

        for b in range(4):
            pltpu.make_async_copy(rows[b], agg_sh.at[dst_v.at[0]],
                                  ssems[b]).wait()

        def drain_deg(i, carry):
            pltpu.make_async_copy(ones_v, deg_sh.at[dst_v.at[0]],
                                  dsem).wait()
            return carry

        lax.fori_loop(0, _CH // 2, drain_deg, 0)
        plsc.subcore_barrier()

        base = c * _NP + s * _RPT
        pltpu.sync_copy(agg_sh.at[pl.ds(s * _RPT, _RPT)],
                        agg_hbm.at[pl.ds(base, _RPT)])
        pltpu.sync_copy(deg_sh.at[pl.ds(s * _RPT, _RPT)],
                        deg_hbm.at[pl.ds(base, _RPT)])

    return body(hstack, srcx, dst2, zD, z1, o1)


def _post_tc(agg2, deg2, cid, W_ops, alphas, W_out):
    def body(a_ref, d_ref, c_ref, w_ref, al_ref, wo_ref, o_ref):
        deg = jnp.maximum(d_ref[:_N, 0:1] + d_ref[_NP:_NP + _N, 0:1], 1.0)
        agg = jnp.concatenate(
            [a_ref[:_N, :], a_ref[_NP:_NP + _N, :]], axis=1) / deg
        cid = c_ref[...]
        res = jnp.zeros((_N, _H), jnp.float32)
        for k in range(_K):
            hk = jnp.dot(agg, w_ref[k], preferred_element_type=jnp.float32)
            mask = (cid == k).astype(jnp.float32)
            res = res + al_ref[0, k] * (mask * hk)
        o_ref[...] = jnp.dot(res, wo_ref[...],
                             preferred_element_type=jnp.float32)

    return pl.pallas_call(
        body,
        out_shape=jax.ShapeDtypeStruct((_N, _CLS), jnp.float32),
    )(agg2, deg2, cid, W_ops, alphas.reshape(1, _K), W_out)


def kernel(x, edge_index, cluster_ids, W_pre, b_pre, W_ops, alphas, W_out):
    x = x.astype(jnp.float32)
    ei = edge_index.astype(jnp.int32)
    src2 = ei[0].reshape(_ROWS_TOTAL, _C)
    srcx = jnp.concatenate([src2, src2 + _N], axis=0)
    dst2 = ei[1].reshape(_ROWS_TOTAL, _C)
    cid = cluster_ids.astype(jnp.int32).reshape(_N, 1)
    zD = jnp.zeros((_RPT, _HH), jnp.float32)
    z1 = jnp.zeros((_RPT, 16), jnp.float32)
    o1 = jnp.ones((_C, 16), jnp.float32)

    hstack = _pre_tc(x, W_pre, b_pre)
    agg2, deg2 = _sc_agg(hstack, srcx, dst2, zD, z1, o1)
    return _post_tc(agg2, deg2, cid, W_ops, alphas, W_out)
